# baseline (speedup 1.0000x reference)
import functools

import jax
import jax.numpy as jnp
from jax import lax
from jax.experimental import pallas as pl
from jax.experimental.pallas import tpu as pltpu
from jax.experimental.pallas import tpu_sc as plsc

NC = 2
NS = 16
NW = NC * NS
LANES = 16
CH = 128
FEAT = 10

DNT = (((1,), (0,)), ((), ()))
F32 = jnp.float32
HI = lax.Precision.HIGHEST


def _dotn(a, b):
    return lax.dot_general(a, b, DNT, preferred_element_type=F32, precision=HI)


def _proj1_body(x_ref, kl_ref, cl_ref, kr_ref, cr_ref, xl_ref, xr_ref):
    x = x_ref[...]
    xl_ref[...] = _dotn(x, kl_ref[...]) + cl_ref[...]
    xr_ref[...] = _dotn(x, kr_ref[...]) + cr_ref[...]


def _pre_body(x_ref, kl2_ref, cl2_ref, kr2_ref, cr2_ref,
              kl3_ref, cl3_ref, kr3_ref, cr3_ref, kv_ref, cv_ref,
              pl2_ref, pr2_ref, pl3_ref, pr3_ref, pv_ref):
    x = x_ref[...]
    pl2_ref[...] = _dotn(x, kl2_ref[...]) + cl2_ref[...]
    pr2_ref[...] = _dotn(x, kr2_ref[...]) + cr2_ref[...]
    pl3_ref[...] = _dotn(x, kl3_ref[...]) + cl3_ref[...]
    pr3_ref[...] = _dotn(x, kr3_ref[...]) + cr3_ref[...]
    pv_ref[...] = _dotn(x, kv_ref[...]) + cv_ref[...]


def _norm(pa_ref, pb_ref, s10_ref, bias_ref):
    num = pa_ref[...] + pb_ref[...]
    den = _dotn(num, s10_ref[...])
    return jnp.maximum(num / (den + 1e-16) + bias_ref[...], 0.0)


def _finproj_body(pa_ref, pb_ref, s10_ref, bias_ref, pre_l_ref, pre_r_ref,
                  wl_ref, wr_ref, xl_ref, xr_ref):
    x = _norm(pa_ref, pb_ref, s10_ref, bias_ref)
    xl_ref[...] = _dotn(x, wl_ref[...]) + pre_l_ref[...]
    xr_ref[...] = _dotn(x, wr_ref[...]) + pre_r_ref[...]


def _value_body(ng, n, r8, pa_ref, pb_ref, s10_ref, bias_ref, pv_ref,
                wv_ref, w2_ref, b2_ref, out_ref, acc_ref):
    i = pl.program_id(0)

    @pl.when(i == 0)
    def _():
        acc_ref[0, 0] = 0.0

    x = _norm(pa_ref, pb_ref, s10_ref, bias_ref)
    h = jnp.maximum(_dotn(x, wv_ref[...]) + pv_ref[...], 0.0)
    row = i * r8 + lax.broadcasted_iota(jnp.int32, (r8, 128), 0)
    h = jnp.where(row < ng, h, 0.0)
    acc_ref[0, 0] += jnp.sum(h * w2_ref[...])

    @pl.when(i == pl.num_programs(0) - 1)
    def _():
        out_ref[...] = jnp.tanh(acc_ref[0, 0] / n + b2_ref[0, 0]) \
            * jnp.ones((1, 1), F32)


def _xm_body(ga_ref, gb_ref, s10_ref, bias_ref, xm_ref):
    xm_ref[...] = _norm(ga_ref, gb_ref, s10_ref, bias_ref)


def _move_mlp_body(fa_ref, fd_ref, mt_ref, waaa_ref, baaa_ref, wccc_ref,
                   bccc_ref, w2_ref, zp_ref):
    dn = (((1,), (1,)), ((), ()))
    dot = lambda a, b: lax.dot_general(a, b, dn, preferred_element_type=F32,
                                       precision=HI)
    a = jnp.maximum(dot(fa_ref[...], waaa_ref[...]) + baaa_ref[...], 0.0)
    d = jnp.maximum(dot(fd_ref[...], wccc_ref[...]) + bccc_ref[...], 0.0)
    asdf = jnp.where(mt_ref[...] == 1, a, d)
    zp_ref[...] = dot(asdf, w2_ref[...])


def _move_pool_body(z_ref, pi_ref, out_ref):
    z = z_ref[...]
    m = jnp.max(z, axis=1, keepdims=True)
    e = jnp.exp(z - m)
    p = jnp.sum(e * pi_ref[...], axis=1, keepdims=True) / \
        jnp.sum(e, axis=1, keepdims=True)
    pm = jnp.max(p)
    lse = jnp.log(jnp.sum(jnp.exp(p - pm))) + pm
    out_ref[...] = p - lse


def _edge_pass(np_, et_pad, ne, n):
    per_w = et_pad // NW
    n_chunks = per_w // CH
    ne_ch = ne // CH
    rows_per_tile = np_ // NS
    n_zero = rows_per_tile // CH
    mesh = plsc.VectorSubcoreMesh(core_axis_name="c", subcore_axis_name="s")
    assert n_chunks % 2 == 0

    nd = 4
    assert n_chunks % nd == 0
    scratch = [
        pltpu.VMEM_SHARED((np_, 16), F32),
        pltpu.VMEM((2 * nd, CH), jnp.int32),
        pltpu.VMEM((2 * nd, CH), jnp.int32),
    ]
    scratch += [pltpu.VMEM((CH, 16), F32)] * (3 * nd)
    scratch += [pltpu.VMEM((16,), F32)]
    scratch += [pltpu.SemaphoreType.DMA] * (3 * nd)

    @functools.partial(
        pl.kernel,
        out_type=(jax.ShapeDtypeStruct((np_, 16), F32),
                  jax.ShapeDtypeStruct((np_, 16), F32)),
        mesh=mesh,
        scratch_types=scratch,
        compiler_params=pltpu.CompilerParams(needs_layout_passes=False,
                                             use_tc_tiling_on_sc=False),
    )
    def k(xlp, xrp, er, attp, out_a, out_b, accum, srcr, dstr, *bufs):
        xs = bufs[0:nd]
        xd = bufs[nd:2 * nd]
        ob = bufs[2 * nd:3 * nd]
        attv = bufs[3 * nd]
        isem = bufs[3 * nd + 1:3 * nd + 1 + nd]
        gsem = bufs[3 * nd + 1 + nd:3 * nd + 1 + 2 * nd]
        scsem = bufs[3 * nd + 1 + 2 * nd:3 * nd + 1 + 3 * nd]
        c = lax.axis_index("c")
        s = lax.axis_index("s")
        wid = s * NC + c
        tile_base = s * rows_per_tile

        def _zrow(i, _):
            ob[0][i] = jnp.zeros((16,), F32)
            return 0
        lax.fori_loop(0, CH, _zrow, 0)

        def _zacc(i, _):
            pltpu.async_copy(ob[0],
                             accum.at[pl.ds(tile_base + i * CH, CH)], scsem[0])
            return 0
        lax.fori_loop(0, n_zero, _zacc, 0)

        def _zw(i, _):
            pltpu.make_async_copy(
                ob[0], accum.at[pl.ds(tile_base, CH)], scsem[0]).wait()
            return 0
        lax.fori_loop(0, n_zero, _zw, 0)

        pltpu.sync_copy(attp, attv)
        plsc.subcore_barrier()

        attvec = attv[...]
        rmask = 2 * nd - 1

        def _idx(t, sem):
            tg = wid * n_chunks + t
            slot = t & rmask

            @pl.when(tg < ne_ch)
            def _():
                pltpu.async_copy(er.at[0, tg], srcr.at[slot], sem)
                pltpu.async_copy(er.at[1, tg], dstr.at[slot], sem)

            @pl.when(tg >= ne_ch)
            def _():
                base = tg * CH - ne
                for gi in range(CH // LANES):
                    vec = jnp.minimum(
                        lax.iota(jnp.int32, 16) + (base + gi * LANES), n)
                    srcr[slot, pl.ds(gi * LANES, LANES)] = vec
                    dstr[slot, pl.ds(gi * LANES, LANES)] = vec

        def _idx_wait(t, sem):
            tg = wid * n_chunks + t

            @pl.when(tg < ne_ch)
            def _():
                pltpu.make_async_copy(er.at[0, 0], srcr.at[0], sem).wait()
                pltpu.make_async_copy(er.at[1, 0], dstr.at[0], sem).wait()

        def _gather(t, b, sem):
            pltpu.async_copy(xlp.at[srcr.at[t & rmask]], xs[b], sem)
            pltpu.async_copy(xrp.at[dstr.at[t & rmask]], xd[b], sem)

        def _gwait(t, b, sem):
            pltpu.make_async_copy(xlp.at[srcr.at[t & rmask]], xs[b],
                                  sem).wait()
            pltpu.make_async_copy(xrp.at[dstr.at[t & rmask]], xd[b],
                                  sem).wait()

        def _compute(b):
            for ei in range(CH):
                a = xs[b][ei]
                sj = a + xd[b][ei]
                lj = jnp.maximum(sj, 0.2 * sj)
                lg = jnp.sum(lj * attvec)
                exb = jnp.exp(jnp.broadcast_to(lg, (16,)))
                ob[b][ei] = exb * a

        def _scwait(t, b, sem):
            pltpu.make_async_copy(ob[b], accum.at[dstr.at[t & rmask]],
                                  sem).wait()

        for q in range(nd):
            _idx(q, isem[q])
        for q in range(2):
            _idx_wait(q, isem[q])
            _gather(q, q, gsem[q])

        def _quad(tt, _):
            for q in range(nd):
                t = tt * nd + q

                @pl.when(tt > 0)
                def _():
                    _scwait(t - nd, q, scsem[q])

                @pl.when(t + nd < n_chunks)
                def _():
                    _idx(t + nd, isem[q])

                @pl.when(t + 2 < n_chunks)
                def _():
                    _idx_wait(t + 2, isem[(q + 2) % nd])
                    _gather(t + 2, (q + 2) % nd, gsem[(q + 2) % nd])
                _gwait(t, q, gsem[q])
                _compute(q)
                pltpu.async_copy(ob[q], accum.at[dstr.at[t & rmask]],
                                 scsem[q], add=True)
            return 0

        lax.fori_loop(0, n_chunks // nd, _quad, 0)
        for q in range(nd):
            _scwait(0, q, scsem[q])
        plsc.subcore_barrier()

        @pl.when(c == 0)
        def _():
            pltpu.sync_copy(accum.at[pl.ds(tile_base, rows_per_tile)],
                            out_a.at[pl.ds(tile_base, rows_per_tile)])

        @pl.when(c == 1)
        def _():
            pltpu.sync_copy(accum.at[pl.ds(tile_base, rows_per_tile)],
                            out_b.at[pl.ds(tile_base, rows_per_tile)])

    return k


def _move_gather(np_, nidx):
    per_w = nidx // NW
    mesh = plsc.VectorSubcoreMesh(core_axis_name="c", subcore_axis_name="s")

    @functools.partial(
        pl.kernel,
        out_type=(
            jax.ShapeDtypeStruct((nidx, 16), F32),
            jax.ShapeDtypeStruct((nidx, 16), F32),
            jax.ShapeDtypeStruct((nidx, 16), F32),
        ),
        mesh=mesh,
        scratch_types=[
            pltpu.VMEM((per_w,), jnp.int32),
            pltpu.VMEM((per_w, 16), F32),
            pltpu.SemaphoreType.DMA,
        ],
        compiler_params=pltpu.CompilerParams(needs_layout_passes=False,
                                             use_tc_tiling_on_sc=False),
    )
    def k(ta, tb, tx, mi, oa, ob, ox, idxb, rows, sem):
        c = lax.axis_index("c")
        s = lax.axis_index("s")
        wid = s * NC + c
        base = wid * per_w
        pltpu.sync_copy(mi.at[pl.ds(base, per_w)], idxb)
        pltpu.async_copy(ta.at[idxb], rows, sem).wait()
        pltpu.sync_copy(rows, oa.at[pl.ds(base, per_w)])
        pltpu.async_copy(tb.at[idxb], rows, sem).wait()
        pltpu.sync_copy(rows, ob.at[pl.ds(base, per_w)])
        pltpu.async_copy(tx.at[idxb], rows, sem).wait()
        pltpu.sync_copy(rows, ox.at[pl.ds(base, per_w)])

    return k


def kernel(x1, x2, edges, move_src, move_dst, move_type, move_armies,
           Wl1, bl1, Wr1, br1, att1, bias1,
           Wl2, bl2, Wr2, br2, att2, bias2,
           Wl3, bl3, Wr3, br3, att3, bias3,
           Wlin, blin, Wlin2, blin2, Waaa, baaa, Wccc, bccc, Watt, batt,
           Wpi, bpi):
    n = x1.shape[0]
    e = edges.shape[1]
    m, l = move_src.shape
    r8 = 3200
    blk = 8 * r8
    np_ = ((n + blk - 1) // blk) * blk
    ngrp = np_ // 8
    grid = ngrp // r8

    et = e + n
    et_pad = ((et + NW * CH - 1) // (NW * CH)) * (NW * CH)
    er = edges.reshape(2, e // CH, CH)

    x1w = jnp.pad(x1.reshape(n // 8, 120), ((0, ngrp - n // 8), (0, 0)))
    x1g = jnp.pad(x1, ((0, np_ - n), (0, 1)))
    e10 = (jnp.arange(16) == FEAT).astype(F32).reshape(1, 16)
    eye8 = jnp.eye(8, dtype=F32)

    def cpad(b, carrier):
        out = jnp.pad(b, (0, 16 - b.shape[0])).reshape(1, 16)
        return out + e10 if carrier else out

    def c128(b, carrier):
        return jnp.tile(cpad(b, carrier), (1, 8))

    def ext(w, kdim):
        return jnp.pad(w.T, ((0, kdim - w.shape[1]), (0, 16 - w.shape[0])))

    def kron8(w, kdim):
        return jnp.kron(eye8, ext(w, kdim))

    sel10 = jnp.zeros((16, 16), F32).at[FEAT, :].set(1.0)
    s10 = jnp.kron(eye8, sel10)

    wspec = lambda shp: pl.BlockSpec(shp, lambda i: (0, 0))
    gspec = pl.BlockSpec((r8, 128), lambda i: (i, 0))
    xspec = pl.BlockSpec((r8, 120), lambda i: (i, 0))
    g16 = jax.ShapeDtypeStruct((ngrp, 128), F32)

    xlw1, xrw1 = pl.pallas_call(
        _proj1_body,
        grid=(grid,),
        in_specs=[xspec, wspec((120, 128)), wspec((1, 128)),
                  wspec((120, 128)), wspec((1, 128))],
        out_specs=[gspec, gspec],
        out_shape=[g16, g16],
    )(x1w, kron8(Wl1, 15), c128(bl1, True),
      kron8(Wr1, 15), c128(br1, False))

    cv = blin.reshape(1, 15) + x2 @ Wlin[:, FEAT + 15:].T
    pl2, pr2, pl3, pr3, pvw = pl.pallas_call(
        _pre_body,
        grid=(grid,),
        in_specs=[xspec] + [wspec((120, 128)), wspec((1, 128))] * 4
        + [wspec((120, 128)), wspec((1, 128))],
        out_specs=[gspec] * 5,
        out_shape=[g16] * 5,
    )(x1w,
      kron8(Wl2[:, FEAT:], 15), c128(bl2, True),
      kron8(Wr2[:, FEAT:], 15), c128(br2, False),
      kron8(Wl3[:, FEAT:], 15), c128(bl3, True),
      kron8(Wr3[:, FEAT:], 15), c128(br3, False),
      kron8(Wlin[:, FEAT:FEAT + 15], 15), c128(cv[0], False))

    edge_k = _edge_pass(np_, et_pad, e, n)

    def as16(aw):
        return aw.reshape(np_, 16)

    def asw(a):
        return a.reshape(ngrp, 128)

    pa1, pb1 = edge_k(as16(xlw1), as16(xrw1), er, jnp.pad(att1, (0, 6)))

    def finproj(pa, pb, bias, pre_l, pre_r, wl, wr):
        return pl.pallas_call(
            _finproj_body,
            grid=(grid,),
            in_specs=[gspec, gspec, wspec((128, 128)), wspec((1, 128)),
                      gspec, gspec, wspec((128, 128)), wspec((128, 128))],
            out_specs=[gspec, gspec],
            out_shape=[g16, g16],
        )(asw(pa), asw(pb), s10, c128(bias, False), pre_l, pre_r,
          kron8(wl[:, :FEAT], 16), kron8(wr[:, :FEAT], 16))

    xlw2, xrw2 = finproj(pa1, pb1, bias1, pl2, pr2, Wl2, Wr2)
    pa2, pb2 = edge_k(as16(xlw2), as16(xrw2), er, jnp.pad(att2, (0, 6)))

    xlw3, xrw3 = finproj(pa2, pb2, bias2, pl3, pr3, Wl3, Wr3)
    pa3, pb3 = edge_k(as16(xlw3), as16(xrw3), er, jnp.pad(att3, (0, 6)))

    vout = pl.pallas_call(
        functools.partial(_value_body, n // 8, n, r8),
        grid=(grid,),
        in_specs=[gspec, gspec, wspec((128, 128)), wspec((1, 128)), gspec,
                  wspec((128, 128)), wspec((1, 128)),
                  pl.BlockSpec(memory_space=pltpu.SMEM)],
        out_specs=pl.BlockSpec((1, 1), lambda i: (0, 0)),
        out_shape=jax.ShapeDtypeStruct((1, 1), F32),
        scratch_shapes=[pltpu.SMEM((1, 1), F32)],
    )(asw(pa3), asw(pb3), s10, c128(bias3, False), pvw,
      kron8(Wlin[:, :FEAT], 16), c128(Wlin2[0], False),
      blin2.reshape(1, 1))
    v = vout[0, 0]

    nidx = 2 * m * l
    midx = jnp.concatenate([move_src.reshape(-1), move_dst.reshape(-1)])
    gk = _move_gather(np_, nidx)
    ga, gb, gx1 = gk(pa3, pb3, x1g, midx)

    nw8 = nidx // 8
    xm = pl.pallas_call(
        _xm_body,
        in_specs=[pl.BlockSpec((nw8, 128), lambda: (0, 0))] * 2
        + [pl.BlockSpec((128, 128), lambda: (0, 0)),
           pl.BlockSpec((1, 128), lambda: (0, 0))],
        out_specs=pl.BlockSpec((nw8, 128), lambda: (0, 0)),
        out_shape=jax.ShapeDtypeStruct((nw8, 128), F32),
    )(ga.reshape(nw8, 128), gb.reshape(nw8, 128), s10,
      c128(bias3, False)).reshape(nidx, 16)

    nm = m * l
    xs = xm[:nm, :FEAT]
    xd = xm[nm:, :FEAT]
    x1s = gx1[:nm, :15]
    x1d = gx1[nm:, :15]
    armies = move_armies.reshape(nm, 1)
    extra = 0.6 * armies - 0.7 * (x1d[:, 3:4] + x1d[:, 4:5])
    f_att = jnp.concatenate(
        [xs, xd, x1s[:, 3:], x1d[:, 1:], armies, extra], axis=1)
    f_dep = jnp.concatenate([xs, x1s[:, 3:], armies], axis=1)
    mtb = jnp.broadcast_to(move_type.reshape(nm, 1), (nm, 20))
    w2 = jnp.concatenate([Watt, Wpi], axis=0)

    zp = pl.pallas_call(
        _move_mlp_body,
        out_shape=jax.ShapeDtypeStruct((nm, 2), F32),
    )(f_att, f_dep, mtb, Waaa, baaa.reshape(1, 20), Wccc, bccc.reshape(1, 20),
      w2)

    logp = pl.pallas_call(
        _move_pool_body,
        out_shape=jax.ShapeDtypeStruct((m, 1), F32),
    )(zp[:, 0].reshape(m, l), zp[:, 1].reshape(m, l))

    return (v, logp[:, 0])


# speedup vs baseline: 1.0369x; 1.0369x over previous
import functools

import jax
import jax.numpy as jnp
from jax import lax
from jax.experimental import pallas as pl
from jax.experimental.pallas import tpu as pltpu
from jax.experimental.pallas import tpu_sc as plsc

NC = 2
NS = 16
NW = NC * NS
LANES = 16
CH = 256
GR = 1
FEAT = 10

DNT = (((1,), (0,)), ((), ()))
F32 = jnp.float32
HI = lax.Precision.HIGHEST


def _dotn(a, b):
    return lax.dot_general(a, b, DNT, preferred_element_type=F32, precision=HI)


def _proj1_body(x_ref, kl_ref, cl_ref, kr_ref, cr_ref, xl_ref, xr_ref):
    x = x_ref[...]
    xl_ref[...] = _dotn(x, kl_ref[...]) + cl_ref[...]
    xr_ref[...] = _dotn(x, kr_ref[...]) + cr_ref[...]


def _pre_body(x_ref, kl2_ref, cl2_ref, kr2_ref, cr2_ref,
              kl3_ref, cl3_ref, kr3_ref, cr3_ref, kv_ref, cv_ref,
              pl2_ref, pr2_ref, pl3_ref, pr3_ref, pv_ref):
    x = x_ref[...]
    pl2_ref[...] = _dotn(x, kl2_ref[...]) + cl2_ref[...]
    pr2_ref[...] = _dotn(x, kr2_ref[...]) + cr2_ref[...]
    pl3_ref[...] = _dotn(x, kl3_ref[...]) + cl3_ref[...]
    pr3_ref[...] = _dotn(x, kr3_ref[...]) + cr3_ref[...]
    pv_ref[...] = _dotn(x, kv_ref[...]) + cv_ref[...]


def _norm(pa_ref, pb_ref, s10_ref, bias_ref):
    num = pa_ref[...] + pb_ref[...]
    den = _dotn(num, s10_ref[...])
    return jnp.maximum(num / (den + 1e-16) + bias_ref[...], 0.0)


def _finproj_body(pa_ref, pb_ref, s10_ref, bias_ref, pre_l_ref, pre_r_ref,
                  wl_ref, wr_ref, xl_ref, xr_ref):
    x = _norm(pa_ref, pb_ref, s10_ref, bias_ref)
    xl_ref[...] = _dotn(x, wl_ref[...]) + pre_l_ref[...]
    xr_ref[...] = _dotn(x, wr_ref[...]) + pre_r_ref[...]


def _value_body(ng, n, r8, pa_ref, pb_ref, s10_ref, bias_ref, pv_ref,
                wv_ref, w2_ref, b2_ref, out_ref, acc_ref):
    i = pl.program_id(0)

    @pl.when(i == 0)
    def _():
        acc_ref[0, 0] = 0.0

    x = _norm(pa_ref, pb_ref, s10_ref, bias_ref)
    h = jnp.maximum(_dotn(x, wv_ref[...]) + pv_ref[...], 0.0)
    row = i * r8 + lax.broadcasted_iota(jnp.int32, (r8, 128), 0)
    h = jnp.where(row < ng, h, 0.0)
    acc_ref[0, 0] += jnp.sum(h * w2_ref[...])

    @pl.when(i == pl.num_programs(0) - 1)
    def _():
        out_ref[...] = jnp.tanh(acc_ref[0, 0] / n + b2_ref[0, 0]) \
            * jnp.ones((1, 1), F32)


def _xm_body(ga_ref, gb_ref, s10_ref, bias_ref, xm_ref):
    xm_ref[...] = _norm(ga_ref, gb_ref, s10_ref, bias_ref)


def _move_mlp_body(fa_ref, fd_ref, mt_ref, waaa_ref, baaa_ref, wccc_ref,
                   bccc_ref, w2_ref, zp_ref):
    dn = (((1,), (1,)), ((), ()))
    dot = lambda a, b: lax.dot_general(a, b, dn, preferred_element_type=F32,
                                       precision=HI)
    a = jnp.maximum(dot(fa_ref[...], waaa_ref[...]) + baaa_ref[...], 0.0)
    d = jnp.maximum(dot(fd_ref[...], wccc_ref[...]) + bccc_ref[...], 0.0)
    asdf = jnp.where(mt_ref[...] == 1, a, d)
    zp_ref[...] = dot(asdf, w2_ref[...])


def _move_pool_body(z_ref, pi_ref, out_ref):
    z = z_ref[...]
    m = jnp.max(z, axis=1, keepdims=True)
    e = jnp.exp(z - m)
    p = jnp.sum(e * pi_ref[...], axis=1, keepdims=True) / \
        jnp.sum(e, axis=1, keepdims=True)
    pm = jnp.max(p)
    lse = jnp.log(jnp.sum(jnp.exp(p - pm))) + pm
    out_ref[...] = p - lse


def _edge_pass(np_, et_pad, ne, n):
    sch = GR * CH
    per_w = et_pad // NW
    n_chunks = per_w // sch
    ne_ch = ne // sch
    rows_per_tile = np_ // NS
    n_zero = rows_per_tile // CH
    mesh = plsc.VectorSubcoreMesh(core_axis_name="c", subcore_axis_name="s")
    assert n_chunks % 2 == 0 and ne % sch == 0

    scratch = [
        pltpu.VMEM_SHARED((np_, 16), F32),
        pltpu.VMEM((4, CH), jnp.int32),
        pltpu.VMEM((4, CH), jnp.int32),
    ]
    scratch += [pltpu.VMEM((CH, 16), F32)] * 6
    scratch += [pltpu.VMEM((16,), F32)]
    scratch += [pltpu.SemaphoreType.DMA] * 6

    @functools.partial(
        pl.kernel,
        out_type=(jax.ShapeDtypeStruct((np_, 16), F32),
                  jax.ShapeDtypeStruct((np_, 16), F32)),
        mesh=mesh,
        scratch_types=scratch,
        compiler_params=pltpu.CompilerParams(needs_layout_passes=False,
                                             use_tc_tiling_on_sc=False),
    )
    def k(xlp, xrp, er, attp, out_a, out_b, accum, srcr, dstr,
          xs0, xd0, ob0, xs1, xd1, ob1, attv, ia, ib, g0, g1, sc0, sc1):
        c = lax.axis_index("c")
        s = lax.axis_index("s")
        wid = s * NC + c
        tile_base = s * rows_per_tile

        def _zrow(i, _):
            ob0[i] = jnp.zeros((16,), F32)
            return 0
        lax.fori_loop(0, CH, _zrow, 0)

        def _zacc(i, _):
            pltpu.async_copy(
                ob0, accum.at[pl.ds(tile_base + i * CH, CH)], sc0)
            return 0
        lax.fori_loop(0, n_zero, _zacc, 0)

        def _zw(i, _):
            pltpu.make_async_copy(
                ob0, accum.at[pl.ds(tile_base, CH)], sc0).wait()
            return 0
        lax.fori_loop(0, n_zero, _zw, 0)

        pltpu.sync_copy(attp, attv)
        plsc.subcore_barrier()

        attvec = attv[...]

        def _idx(t, sem):
            tg = wid * n_chunks + t
            slot = t & 3

            @pl.when(tg < ne_ch)
            def _():
                pltpu.async_copy(er.at[0, tg], srcr.at[slot], sem)
                pltpu.async_copy(er.at[1, tg], dstr.at[slot], sem)

            @pl.when(tg >= ne_ch)
            def _():
                base = tg * sch - ne
                for gi in range(CH // LANES):
                    vec = jnp.minimum(
                        lax.iota(jnp.int32, 16) + (base + gi * LANES), n)
                    srcr[slot, pl.ds(gi * LANES, LANES)] = vec
                    dstr[slot, pl.ds(gi * LANES, LANES)] = vec

        def _idx_wait(t, sem):
            tg = wid * n_chunks + t

            @pl.when(tg < ne_ch)
            def _():
                pltpu.make_async_copy(er.at[0, 0], srcr.at[0], sem).wait()
                pltpu.make_async_copy(er.at[1, 0], dstr.at[0], sem).wait()

        def _gather(t, xs, xd, sem):
            pltpu.async_copy(xlp.at[srcr.at[t & 3]], xs, sem)
            pltpu.async_copy(xrp.at[dstr.at[t & 3]], xd, sem)

        def _gwait(t, xs, xd, sem):
            pltpu.make_async_copy(xlp.at[srcr.at[t & 3]], xs, sem).wait()
            pltpu.make_async_copy(xrp.at[dstr.at[t & 3]], xd, sem).wait()

        def _compute(xs, xd, ob):
            for ei in range(CH):
                a = xs[ei]
                sj = a + xd[ei]
                lj = jnp.maximum(sj, 0.2 * sj)
                lg = jnp.sum(lj * attvec)
                exb = jnp.exp(jnp.broadcast_to(lg, (16,)))
                ob[ei] = exb * a

        def _scwait(t, ob, sem):
            pltpu.make_async_copy(ob, accum.at[dstr.at[t & 3]], sem).wait()

        _idx(0, ia)
        _idx(1, ib)
        _idx_wait(0, ia)
        _gather(0, xs0, xd0, g0)

        def _pair(tt, _):
            t0 = tt * 2
            t1 = t0 + 1

            @pl.when(tt > 0)
            def _():
                _scwait(t0, ob0, sc0)

            @pl.when(t0 + 2 < n_chunks)
            def _():
                _idx(t0 + 2, ia)
            _idx_wait(t1, ib)
            _gather(t1, xs1, xd1, g1)
            _gwait(t0, xs0, xd0, g0)
            _compute(xs0, xd0, ob0)
            pltpu.async_copy(ob0, accum.at[dstr.at[t0 & 3]], sc0, add=True)

            @pl.when(tt > 0)
            def _():
                _scwait(t1, ob1, sc1)

            @pl.when(t1 + 2 < n_chunks)
            def _():
                _idx(t1 + 2, ib)

            @pl.when(t0 + 2 < n_chunks)
            def _():
                _idx_wait(t0 + 2, ia)
                _gather(t0 + 2, xs0, xd0, g0)
            _gwait(t1, xs1, xd1, g1)
            _compute(xs1, xd1, ob1)
            pltpu.async_copy(ob1, accum.at[dstr.at[t1 & 3]], sc1, add=True)
            return 0

        lax.fori_loop(0, n_chunks // 2, _pair, 0)
        _scwait(0, ob0, sc0)
        _scwait(0, ob1, sc1)
        plsc.subcore_barrier()

        @pl.when(c == 0)
        def _():
            pltpu.sync_copy(accum.at[pl.ds(tile_base, rows_per_tile)],
                            out_a.at[pl.ds(tile_base, rows_per_tile)])

        @pl.when(c == 1)
        def _():
            pltpu.sync_copy(accum.at[pl.ds(tile_base, rows_per_tile)],
                            out_b.at[pl.ds(tile_base, rows_per_tile)])

    return k


def _move_gather(np_, nidx):
    per_w = nidx // NW
    mesh = plsc.VectorSubcoreMesh(core_axis_name="c", subcore_axis_name="s")

    @functools.partial(
        pl.kernel,
        out_type=(
            jax.ShapeDtypeStruct((nidx, 16), F32),
            jax.ShapeDtypeStruct((nidx, 16), F32),
            jax.ShapeDtypeStruct((nidx, 16), F32),
        ),
        mesh=mesh,
        scratch_types=[
            pltpu.VMEM((per_w,), jnp.int32),
            pltpu.VMEM((per_w, 16), F32),
            pltpu.SemaphoreType.DMA,
        ],
        compiler_params=pltpu.CompilerParams(needs_layout_passes=False,
                                             use_tc_tiling_on_sc=False),
    )
    def k(ta, tb, tx, mi, oa, ob, ox, idxb, rows, sem):
        c = lax.axis_index("c")
        s = lax.axis_index("s")
        wid = s * NC + c
        base = wid * per_w
        pltpu.sync_copy(mi.at[pl.ds(base, per_w)], idxb)
        pltpu.async_copy(ta.at[idxb], rows, sem).wait()
        pltpu.sync_copy(rows, oa.at[pl.ds(base, per_w)])
        pltpu.async_copy(tb.at[idxb], rows, sem).wait()
        pltpu.sync_copy(rows, ob.at[pl.ds(base, per_w)])
        pltpu.async_copy(tx.at[idxb], rows, sem).wait()
        pltpu.sync_copy(rows, ox.at[pl.ds(base, per_w)])

    return k


def kernel(x1, x2, edges, move_src, move_dst, move_type, move_armies,
           Wl1, bl1, Wr1, br1, att1, bias1,
           Wl2, bl2, Wr2, br2, att2, bias2,
           Wl3, bl3, Wr3, br3, att3, bias3,
           Wlin, blin, Wlin2, blin2, Waaa, baaa, Wccc, bccc, Watt, batt,
           Wpi, bpi):
    n = x1.shape[0]
    e = edges.shape[1]
    m, l = move_src.shape
    r8 = 3200
    blk = 8 * r8
    np_ = ((n + blk - 1) // blk) * blk
    ngrp = np_ // 8
    grid = ngrp // r8

    et = e + n
    egr = NW * GR * CH
    et_pad = ((et + egr - 1) // egr) * egr
    er = edges.reshape(2, e // (GR * CH), CH)

    x1w = jnp.pad(x1.reshape(n // 8, 120), ((0, ngrp - n // 8), (0, 0)))
    x1g = jnp.pad(x1, ((0, np_ - n), (0, 1)))
    e10 = (jnp.arange(16) == FEAT).astype(F32).reshape(1, 16)
    eye8 = jnp.eye(8, dtype=F32)

    def cpad(b, carrier):
        out = jnp.pad(b, (0, 16 - b.shape[0])).reshape(1, 16)
        return out + e10 if carrier else out

    def c128(b, carrier):
        return jnp.tile(cpad(b, carrier), (1, 8))

    def ext(w, kdim):
        return jnp.pad(w.T, ((0, kdim - w.shape[1]), (0, 16 - w.shape[0])))

    def kron8(w, kdim):
        return jnp.kron(eye8, ext(w, kdim))

    sel10 = jnp.zeros((16, 16), F32).at[FEAT, :].set(1.0)
    s10 = jnp.kron(eye8, sel10)

    wspec = lambda shp: pl.BlockSpec(shp, lambda i: (0, 0))
    gspec = pl.BlockSpec((r8, 128), lambda i: (i, 0))
    xspec = pl.BlockSpec((r8, 120), lambda i: (i, 0))
    g16 = jax.ShapeDtypeStruct((ngrp, 128), F32)

    xlw1, xrw1 = pl.pallas_call(
        _proj1_body,
        grid=(grid,),
        in_specs=[xspec, wspec((120, 128)), wspec((1, 128)),
                  wspec((120, 128)), wspec((1, 128))],
        out_specs=[gspec, gspec],
        out_shape=[g16, g16],
    )(x1w, kron8(Wl1, 15), c128(bl1, True),
      kron8(Wr1, 15), c128(br1, False))

    cv = blin.reshape(1, 15) + x2 @ Wlin[:, FEAT + 15:].T
    pl2, pr2, pl3, pr3, pvw = pl.pallas_call(
        _pre_body,
        grid=(grid,),
        in_specs=[xspec] + [wspec((120, 128)), wspec((1, 128))] * 4
        + [wspec((120, 128)), wspec((1, 128))],
        out_specs=[gspec] * 5,
        out_shape=[g16] * 5,
    )(x1w,
      kron8(Wl2[:, FEAT:], 15), c128(bl2, True),
      kron8(Wr2[:, FEAT:], 15), c128(br2, False),
      kron8(Wl3[:, FEAT:], 15), c128(bl3, True),
      kron8(Wr3[:, FEAT:], 15), c128(br3, False),
      kron8(Wlin[:, FEAT:FEAT + 15], 15), c128(cv[0], False))

    edge_k = _edge_pass(np_, et_pad, e, n)

    def as16(aw):
        return aw.reshape(np_, 16)

    def asw(a):
        return a.reshape(ngrp, 128)

    pa1, pb1 = edge_k(as16(xlw1), as16(xrw1), er, jnp.pad(att1, (0, 6)))

    def finproj(pa, pb, bias, pre_l, pre_r, wl, wr):
        return pl.pallas_call(
            _finproj_body,
            grid=(grid,),
            in_specs=[gspec, gspec, wspec((128, 128)), wspec((1, 128)),
                      gspec, gspec, wspec((128, 128)), wspec((128, 128))],
            out_specs=[gspec, gspec],
            out_shape=[g16, g16],
        )(asw(pa), asw(pb), s10, c128(bias, False), pre_l, pre_r,
          kron8(wl[:, :FEAT], 16), kron8(wr[:, :FEAT], 16))

    xlw2, xrw2 = finproj(pa1, pb1, bias1, pl2, pr2, Wl2, Wr2)
    pa2, pb2 = edge_k(as16(xlw2), as16(xrw2), er, jnp.pad(att2, (0, 6)))

    xlw3, xrw3 = finproj(pa2, pb2, bias2, pl3, pr3, Wl3, Wr3)
    pa3, pb3 = edge_k(as16(xlw3), as16(xrw3), er, jnp.pad(att3, (0, 6)))

    vout = pl.pallas_call(
        functools.partial(_value_body, n // 8, n, r8),
        grid=(grid,),
        in_specs=[gspec, gspec, wspec((128, 128)), wspec((1, 128)), gspec,
                  wspec((128, 128)), wspec((1, 128)),
                  pl.BlockSpec(memory_space=pltpu.SMEM)],
        out_specs=pl.BlockSpec((1, 1), lambda i: (0, 0)),
        out_shape=jax.ShapeDtypeStruct((1, 1), F32),
        scratch_shapes=[pltpu.SMEM((1, 1), F32)],
    )(asw(pa3), asw(pb3), s10, c128(bias3, False), pvw,
      kron8(Wlin[:, :FEAT], 16), c128(Wlin2[0], False),
      blin2.reshape(1, 1))
    v = vout[0, 0]

    nidx = 2 * m * l
    midx = jnp.concatenate([move_src.reshape(-1), move_dst.reshape(-1)])
    gk = _move_gather(np_, nidx)
    ga, gb, gx1 = gk(pa3, pb3, x1g, midx)

    nw8 = nidx // 8
    xm = pl.pallas_call(
        _xm_body,
        in_specs=[pl.BlockSpec((nw8, 128), lambda: (0, 0))] * 2
        + [pl.BlockSpec((128, 128), lambda: (0, 0)),
           pl.BlockSpec((1, 128), lambda: (0, 0))],
        out_specs=pl.BlockSpec((nw8, 128), lambda: (0, 0)),
        out_shape=jax.ShapeDtypeStruct((nw8, 128), F32),
    )(ga.reshape(nw8, 128), gb.reshape(nw8, 128), s10,
      c128(bias3, False)).reshape(nidx, 16)

    nm = m * l
    xs = xm[:nm, :FEAT]
    xd = xm[nm:, :FEAT]
    x1s = gx1[:nm, :15]
    x1d = gx1[nm:, :15]
    armies = move_armies.reshape(nm, 1)
    extra = 0.6 * armies - 0.7 * (x1d[:, 3:4] + x1d[:, 4:5])
    f_att = jnp.concatenate(
        [xs, xd, x1s[:, 3:], x1d[:, 1:], armies, extra], axis=1)
    f_dep = jnp.concatenate([xs, x1s[:, 3:], armies], axis=1)
    mtb = jnp.broadcast_to(move_type.reshape(nm, 1), (nm, 20))
    w2 = jnp.concatenate([Watt, Wpi], axis=0)

    zp = pl.pallas_call(
        _move_mlp_body,
        out_shape=jax.ShapeDtypeStruct((nm, 2), F32),
    )(f_att, f_dep, mtb, Waaa, baaa.reshape(1, 20), Wccc, bccc.reshape(1, 20),
      w2)

    logp = pl.pallas_call(
        _move_pool_body,
        out_shape=jax.ShapeDtypeStruct((m, 1), F32),
    )(zp[:, 0].reshape(m, l), zp[:, 1].reshape(m, l))

    return (v, logp[:, 0])


# speedup vs baseline: 1.1076x; 1.0681x over previous
import functools

import jax
import jax.numpy as jnp
from jax import lax
from jax.experimental import pallas as pl
from jax.experimental.pallas import tpu as pltpu
from jax.experimental.pallas import tpu_sc as plsc

NC = 2
NS = 16
NW = NC * NS
LANES = 16
CH = 128
GR = 1
FEAT = 10

DNT = (((1,), (0,)), ((), ()))
F32 = jnp.float32
HI = lax.Precision.HIGHEST


def _dotn(a, b):
    return lax.dot_general(a, b, DNT, preferred_element_type=F32, precision=HI)


def _proj1_body(x_ref, kl_ref, cl_ref, kr_ref, cr_ref, xl_ref, xr_ref):
    x = x_ref[...]
    xl_ref[...] = _dotn(x, kl_ref[...]) + cl_ref[...]
    xr_ref[...] = _dotn(x, kr_ref[...]) + cr_ref[...]


def _pre_body(x_ref, kl2_ref, cl2_ref, kr2_ref, cr2_ref,
              kl3_ref, cl3_ref, kr3_ref, cr3_ref, kv_ref, cv_ref,
              pl2_ref, pr2_ref, pl3_ref, pr3_ref, pv_ref):
    x = x_ref[...]
    pl2_ref[...] = _dotn(x, kl2_ref[...]) + cl2_ref[...]
    pr2_ref[...] = _dotn(x, kr2_ref[...]) + cr2_ref[...]
    pl3_ref[...] = _dotn(x, kl3_ref[...]) + cl3_ref[...]
    pr3_ref[...] = _dotn(x, kr3_ref[...]) + cr3_ref[...]
    pv_ref[...] = _dotn(x, kv_ref[...]) + cv_ref[...]


def _norm(pa_ref, pb_ref, s10_ref, bias_ref):
    num = pa_ref[...] + pb_ref[...]
    den = _dotn(num, s10_ref[...])
    return jnp.maximum(num / (den + 1e-16) + bias_ref[...], 0.0)


def _finproj_body(pa_ref, pb_ref, s10_ref, bias_ref, pre_l_ref, pre_r_ref,
                  wl_ref, wr_ref, xl_ref, xr_ref):
    x = _norm(pa_ref, pb_ref, s10_ref, bias_ref)
    xl_ref[...] = _dotn(x, wl_ref[...]) + pre_l_ref[...]
    xr_ref[...] = _dotn(x, wr_ref[...]) + pre_r_ref[...]


def _value_body(ng, n, r8, pa_ref, pb_ref, s10_ref, bias_ref, pv_ref,
                wv_ref, w2_ref, b2_ref, out_ref, acc_ref):
    i = pl.program_id(0)

    @pl.when(i == 0)
    def _():
        acc_ref[0, 0] = 0.0

    x = _norm(pa_ref, pb_ref, s10_ref, bias_ref)
    h = jnp.maximum(_dotn(x, wv_ref[...]) + pv_ref[...], 0.0)
    row = i * r8 + lax.broadcasted_iota(jnp.int32, (r8, 128), 0)
    h = jnp.where(row < ng, h, 0.0)
    acc_ref[0, 0] += jnp.sum(h * w2_ref[...])

    @pl.when(i == pl.num_programs(0) - 1)
    def _():
        out_ref[...] = jnp.tanh(acc_ref[0, 0] / n + b2_ref[0, 0]) \
            * jnp.ones((1, 1), F32)


def _xm_body(ga_ref, gb_ref, s10_ref, bias_ref, xm_ref):
    xm_ref[...] = _norm(ga_ref, gb_ref, s10_ref, bias_ref)


def _move_mlp_body(fa_ref, fd_ref, mt_ref, waaa_ref, baaa_ref, wccc_ref,
                   bccc_ref, w2_ref, zp_ref):
    dn = (((1,), (1,)), ((), ()))
    dot = lambda a, b: lax.dot_general(a, b, dn, preferred_element_type=F32,
                                       precision=HI)
    a = jnp.maximum(dot(fa_ref[...], waaa_ref[...]) + baaa_ref[...], 0.0)
    d = jnp.maximum(dot(fd_ref[...], wccc_ref[...]) + bccc_ref[...], 0.0)
    asdf = jnp.where(mt_ref[...] == 1, a, d)
    zp_ref[...] = dot(asdf, w2_ref[...])


def _move_pool_body(z_ref, pi_ref, out_ref):
    z = z_ref[...]
    m = jnp.max(z, axis=1, keepdims=True)
    e = jnp.exp(z - m)
    p = jnp.sum(e * pi_ref[...], axis=1, keepdims=True) / \
        jnp.sum(e, axis=1, keepdims=True)
    pm = jnp.max(p)
    lse = jnp.log(jnp.sum(jnp.exp(p - pm))) + pm
    out_ref[...] = p - lse


def _edge_pass(np_, et_pad, ne, n):
    sch = GR * CH
    per_w = et_pad // NW
    n_chunks = per_w // sch
    ne_ch = ne // sch
    rows_per_tile = np_ // NS
    n_zero = rows_per_tile // CH
    mesh = plsc.VectorSubcoreMesh(core_axis_name="c", subcore_axis_name="s")
    assert n_chunks % 2 == 0 and ne % sch == 0

    scratch = [
        pltpu.VMEM_SHARED((np_, 16), F32),
        pltpu.VMEM((4, CH), jnp.int32),
        pltpu.VMEM((4, CH), jnp.int32),
    ]
    scratch += [pltpu.VMEM((CH, 16), F32)] * 6
    scratch += [pltpu.VMEM((16,), F32)]
    scratch += [pltpu.SemaphoreType.DMA] * 6

    @functools.partial(
        pl.kernel,
        out_type=(jax.ShapeDtypeStruct((np_, 16), F32),
                  jax.ShapeDtypeStruct((np_, 16), F32)),
        mesh=mesh,
        scratch_types=scratch,
        compiler_params=pltpu.CompilerParams(needs_layout_passes=False,
                                             use_tc_tiling_on_sc=False),
    )
    def k(xlp, xrp, er, attp, out_a, out_b, accum, srcr, dstr,
          xs0, xd0, ob0, xs1, xd1, ob1, attv, ia, ib, g0, g1, sc0, sc1):
        c = lax.axis_index("c")
        s = lax.axis_index("s")
        wid = s * NC + c
        tile_base = s * rows_per_tile

        def _zrow(i, _):
            ob0[i] = jnp.zeros((16,), F32)
            return 0
        lax.fori_loop(0, CH, _zrow, 0)

        def _zacc(i, _):
            pltpu.async_copy(
                ob0, accum.at[pl.ds(tile_base + i * CH, CH)], sc0)
            return 0
        lax.fori_loop(0, n_zero, _zacc, 0)

        def _zw(i, _):
            pltpu.make_async_copy(
                ob0, accum.at[pl.ds(tile_base, CH)], sc0).wait()
            return 0
        lax.fori_loop(0, n_zero, _zw, 0)

        pltpu.sync_copy(attp, attv)
        plsc.subcore_barrier()

        attvec = attv[...]

        def _idx(t, sem):
            tg = wid * n_chunks + t
            slot = t & 3

            @pl.when(tg < ne_ch)
            def _():
                pltpu.async_copy(er.at[0, tg], srcr.at[slot], sem)
                pltpu.async_copy(er.at[1, tg], dstr.at[slot], sem)

            @pl.when(tg >= ne_ch)
            def _():
                base = tg * sch - ne
                for gi in range(CH // LANES):
                    vec = jnp.minimum(
                        lax.iota(jnp.int32, 16) + (base + gi * LANES), n)
                    srcr[slot, pl.ds(gi * LANES, LANES)] = vec
                    dstr[slot, pl.ds(gi * LANES, LANES)] = vec

        def _idx_wait(t, sem):
            tg = wid * n_chunks + t

            @pl.when(tg < ne_ch)
            def _():
                pltpu.make_async_copy(er.at[0, 0], srcr.at[0], sem).wait()
                pltpu.make_async_copy(er.at[1, 0], dstr.at[0], sem).wait()

        def _gather(t, xs, xd, sem):
            pltpu.async_copy(xlp.at[srcr.at[t & 3]], xs, sem)
            pltpu.async_copy(xrp.at[dstr.at[t & 3]], xd, sem)

        def _gwait(t, xs, xd, sem):
            pltpu.make_async_copy(xlp.at[srcr.at[t & 3]], xs, sem).wait()
            pltpu.make_async_copy(xrp.at[dstr.at[t & 3]], xd, sem).wait()

        def _compute(xs, xd, ob):
            for ei in range(CH):
                a = xs[ei]
                sj = a + xd[ei]
                lj = jnp.maximum(sj, 0.2 * sj)
                lg = jnp.sum(lj * attvec)
                exb = jnp.exp(jnp.broadcast_to(lg, (16,)))
                ob[ei] = exb * a

        def _scwait(t, ob, sem):
            pltpu.make_async_copy(ob, accum.at[dstr.at[t & 3]], sem).wait()

        _idx(0, ia)
        _idx(1, ib)
        _idx_wait(0, ia)
        _gather(0, xs0, xd0, g0)

        def _pair(tt, _):
            t0 = tt * 2
            t1 = t0 + 1

            @pl.when(tt > 0)
            def _():
                _scwait(t0, ob0, sc0)

            @pl.when(t0 + 2 < n_chunks)
            def _():
                _idx(t0 + 2, ia)
            _idx_wait(t1, ib)
            _gather(t1, xs1, xd1, g1)
            _gwait(t0, xs0, xd0, g0)
            _compute(xs0, xd0, ob0)
            pltpu.async_copy(ob0, accum.at[dstr.at[t0 & 3]], sc0, add=True)

            @pl.when(tt > 0)
            def _():
                _scwait(t1, ob1, sc1)

            @pl.when(t1 + 2 < n_chunks)
            def _():
                _idx(t1 + 2, ib)

            @pl.when(t0 + 2 < n_chunks)
            def _():
                _idx_wait(t0 + 2, ia)
                _gather(t0 + 2, xs0, xd0, g0)
            _gwait(t1, xs1, xd1, g1)
            _compute(xs1, xd1, ob1)
            pltpu.async_copy(ob1, accum.at[dstr.at[t1 & 3]], sc1, add=True)
            return 0

        lax.fori_loop(0, n_chunks // 2, _pair, 0)
        _scwait(0, ob0, sc0)
        _scwait(0, ob1, sc1)
        plsc.subcore_barrier()

        @pl.when(c == 0)
        def _():
            pltpu.sync_copy(accum.at[pl.ds(tile_base, rows_per_tile)],
                            out_a.at[pl.ds(tile_base, rows_per_tile)])

        @pl.when(c == 1)
        def _():
            pltpu.sync_copy(accum.at[pl.ds(tile_base, rows_per_tile)],
                            out_b.at[pl.ds(tile_base, rows_per_tile)])

    return k


def _move_gather(np_, nidx):
    per_w = nidx // NW
    mesh = plsc.VectorSubcoreMesh(core_axis_name="c", subcore_axis_name="s")

    @functools.partial(
        pl.kernel,
        out_type=(
            jax.ShapeDtypeStruct((nidx, 16), F32),
            jax.ShapeDtypeStruct((nidx, 16), F32),
            jax.ShapeDtypeStruct((nidx, 16), F32),
        ),
        mesh=mesh,
        scratch_types=[
            pltpu.VMEM((per_w,), jnp.int32),
            pltpu.VMEM((per_w, 16), F32),
            pltpu.SemaphoreType.DMA,
        ],
        compiler_params=pltpu.CompilerParams(needs_layout_passes=False,
                                             use_tc_tiling_on_sc=False),
    )
    def k(ta, tb, tx, mi, oa, ob, ox, idxb, rows, sem):
        c = lax.axis_index("c")
        s = lax.axis_index("s")
        wid = s * NC + c
        base = wid * per_w
        pltpu.sync_copy(mi.at[pl.ds(base, per_w)], idxb)
        pltpu.async_copy(ta.at[idxb], rows, sem).wait()
        pltpu.sync_copy(rows, oa.at[pl.ds(base, per_w)])
        pltpu.async_copy(tb.at[idxb], rows, sem).wait()
        pltpu.sync_copy(rows, ob.at[pl.ds(base, per_w)])
        pltpu.async_copy(tx.at[idxb], rows, sem).wait()
        pltpu.sync_copy(rows, ox.at[pl.ds(base, per_w)])

    return k


def kernel(x1, x2, edges, move_src, move_dst, move_type, move_armies,
           Wl1, bl1, Wr1, br1, att1, bias1,
           Wl2, bl2, Wr2, br2, att2, bias2,
           Wl3, bl3, Wr3, br3, att3, bias3,
           Wlin, blin, Wlin2, blin2, Waaa, baaa, Wccc, bccc, Watt, batt,
           Wpi, bpi):
    n = x1.shape[0]
    e = edges.shape[1]
    m, l = move_src.shape
    r8 = 3200
    blk = 8 * r8
    np_ = ((n + blk - 1) // blk) * blk
    ngrp = np_ // 8
    grid = ngrp // r8

    et = e + n
    egr = NW * GR * CH
    et_pad = ((et + egr - 1) // egr) * egr
    er = edges.reshape(2, e // (GR * CH), CH)

    x1w = jnp.pad(x1.reshape(n // 8, 120), ((0, ngrp - n // 8), (0, 0)))
    x1g = jnp.pad(x1, ((0, np_ - n), (0, 1)))
    e10 = (jnp.arange(16) == FEAT).astype(F32).reshape(1, 16)
    eye8 = jnp.eye(8, dtype=F32)

    def cpad(b, carrier):
        out = jnp.pad(b, (0, 16 - b.shape[0])).reshape(1, 16)
        return out + e10 if carrier else out

    def c128(b, carrier):
        return jnp.tile(cpad(b, carrier), (1, 8))

    def ext(w, kdim):
        return jnp.pad(w.T, ((0, kdim - w.shape[1]), (0, 16 - w.shape[0])))

    def kron8(w, kdim):
        return jnp.kron(eye8, ext(w, kdim))

    sel10 = jnp.zeros((16, 16), F32).at[FEAT, :].set(1.0)
    s10 = jnp.kron(eye8, sel10)

    wspec = lambda shp: pl.BlockSpec(shp, lambda i: (0, 0))
    gspec = pl.BlockSpec((r8, 128), lambda i: (i, 0))
    xspec = pl.BlockSpec((r8, 120), lambda i: (i, 0))
    g16 = jax.ShapeDtypeStruct((ngrp, 128), F32)

    xlw1, xrw1 = pl.pallas_call(
        _proj1_body,
        grid=(grid,),
        in_specs=[xspec, wspec((120, 128)), wspec((1, 128)),
                  wspec((120, 128)), wspec((1, 128))],
        out_specs=[gspec, gspec],
        out_shape=[g16, g16],
    )(x1w, kron8(Wl1, 15), c128(bl1, True),
      kron8(Wr1, 15), c128(br1, False))

    cv = blin.reshape(1, 15) + x2 @ Wlin[:, FEAT + 15:].T
    pl2, pr2, pl3, pr3, pvw = pl.pallas_call(
        _pre_body,
        grid=(grid,),
        in_specs=[xspec] + [wspec((120, 128)), wspec((1, 128))] * 4
        + [wspec((120, 128)), wspec((1, 128))],
        out_specs=[gspec] * 5,
        out_shape=[g16] * 5,
    )(x1w,
      kron8(Wl2[:, FEAT:], 15), c128(bl2, True),
      kron8(Wr2[:, FEAT:], 15), c128(br2, False),
      kron8(Wl3[:, FEAT:], 15), c128(bl3, True),
      kron8(Wr3[:, FEAT:], 15), c128(br3, False),
      kron8(Wlin[:, FEAT:FEAT + 15], 15), c128(cv[0], False))

    edge_k = _edge_pass(np_, et_pad, e, n)

    def as16(aw):
        return aw.reshape(np_, 16)

    def asw(a):
        return a.reshape(ngrp, 128)

    pa1, pb1 = edge_k(as16(xlw1), as16(xrw1), er, jnp.pad(att1, (0, 6)))

    def finproj(pa, pb, bias, pre_l, pre_r, wl, wr):
        return pl.pallas_call(
            _finproj_body,
            grid=(grid,),
            in_specs=[gspec, gspec, wspec((128, 128)), wspec((1, 128)),
                      gspec, gspec, wspec((128, 128)), wspec((128, 128))],
            out_specs=[gspec, gspec],
            out_shape=[g16, g16],
        )(asw(pa), asw(pb), s10, c128(bias, False), pre_l, pre_r,
          kron8(wl[:, :FEAT], 16), kron8(wr[:, :FEAT], 16))

    xlw2, xrw2 = finproj(pa1, pb1, bias1, pl2, pr2, Wl2, Wr2)
    pa2, pb2 = edge_k(as16(xlw2), as16(xrw2), er, jnp.pad(att2, (0, 6)))

    xlw3, xrw3 = finproj(pa2, pb2, bias2, pl3, pr3, Wl3, Wr3)
    pa3, pb3 = edge_k(as16(xlw3), as16(xrw3), er, jnp.pad(att3, (0, 6)))

    vout = pl.pallas_call(
        functools.partial(_value_body, n // 8, n, r8),
        grid=(grid,),
        in_specs=[gspec, gspec, wspec((128, 128)), wspec((1, 128)), gspec,
                  wspec((128, 128)), wspec((1, 128)),
                  pl.BlockSpec(memory_space=pltpu.SMEM)],
        out_specs=pl.BlockSpec((1, 1), lambda i: (0, 0)),
        out_shape=jax.ShapeDtypeStruct((1, 1), F32),
        scratch_shapes=[pltpu.SMEM((1, 1), F32)],
    )(asw(pa3), asw(pb3), s10, c128(bias3, False), pvw,
      kron8(Wlin[:, :FEAT], 16), c128(Wlin2[0], False),
      blin2.reshape(1, 1))
    v = vout[0, 0]

    nidx = 2 * m * l
    midx = jnp.concatenate([move_src.reshape(-1), move_dst.reshape(-1)])
    gk = _move_gather(np_, nidx)
    ga, gb, gx1 = gk(pa3, pb3, x1g, midx)

    nw8 = nidx // 8
    xm = pl.pallas_call(
        _xm_body,
        in_specs=[pl.BlockSpec((nw8, 128), lambda: (0, 0))] * 2
        + [pl.BlockSpec((128, 128), lambda: (0, 0)),
           pl.BlockSpec((1, 128), lambda: (0, 0))],
        out_specs=pl.BlockSpec((nw8, 128), lambda: (0, 0)),
        out_shape=jax.ShapeDtypeStruct((nw8, 128), F32),
    )(ga.reshape(nw8, 128), gb.reshape(nw8, 128), s10,
      c128(bias3, False)).reshape(nidx, 16)

    nm = m * l
    xs = xm[:nm, :FEAT]
    xd = xm[nm:, :FEAT]
    x1s = gx1[:nm, :15]
    x1d = gx1[nm:, :15]
    armies = move_armies.reshape(nm, 1)
    extra = 0.6 * armies - 0.7 * (x1d[:, 3:4] + x1d[:, 4:5])
    f_att = jnp.concatenate(
        [xs, xd, x1s[:, 3:], x1d[:, 1:], armies, extra], axis=1)
    f_dep = jnp.concatenate([xs, x1s[:, 3:], armies], axis=1)
    mtb = jnp.broadcast_to(move_type.reshape(nm, 1), (nm, 20))
    w2 = jnp.concatenate([Watt, Wpi], axis=0)

    zp = pl.pallas_call(
        _move_mlp_body,
        out_shape=jax.ShapeDtypeStruct((nm, 2), F32),
    )(f_att, f_dep, mtb, Waaa, baaa.reshape(1, 20), Wccc, bccc.reshape(1, 20),
      w2)

    logp = pl.pallas_call(
        _move_pool_body,
        out_shape=jax.ShapeDtypeStruct((m, 1), F32),
    )(zp[:, 0].reshape(m, l), zp[:, 1].reshape(m, l))

    return (v, logp[:, 0])
